# baseline (speedup 1.0000x reference)
import functools

import jax
import jax.numpy as jnp
from jax import lax
from jax.experimental import pallas as pl
from jax.experimental.pallas import tpu as pltpu
from jax.experimental.pallas import tpu_sc as plsc

P = 4194304
NC = 2
NS = 16
L = 16
M = 8
SHIFT = 23 - M
NBINS = 255 << M
HSIZE = 256 << M
SEC = 2 * HSIZE
SECLOG = SEC.bit_length() - 1
GBASE = NBINS
SH = P // (NC * NS)
W = 8192
NWIN = SH // W
CHUNK = HSIZE // NS


def _k1_body(out_hbm, tgt_hbm, cnt_out, sum_out,
             o_v0, o_v1, t_v0, t_v1, idx_v0, idx_v1, val_v0, val_v1,
             ones_v, zbuf, gbuf, cnt_sh, sum_sh, sem_in, sem_sc):
    o_v = (o_v0, o_v1)
    t_v = (t_v0, t_v1)
    idx_v = (idx_v0, idx_v1)
    val_v = (val_v0, val_v1)
    cid = lax.axis_index("c")
    sid = lax.axis_index("s")
    wid = cid * NS + sid
    zero16 = jnp.zeros((L,), jnp.float32)
    one16 = jnp.ones((L,), jnp.float32)

    @pl.loop(0, W // L)
    def _fill(j):
        ones_v[pl.ds(j * L, L)] = one16

    @pl.loop(0, zbuf.shape[0] // L)
    def _zfill(j):
        zbuf[pl.ds(j * L, L)] = zero16

    zslice = HSIZE // NS
    nz = zslice // zbuf.shape[0]
    @pl.loop(0, 2 * nz)
    def _zero(j):
        sec = j // nz
        base = sec * SEC + sid * zslice + (j - sec * nz) * zbuf.shape[0]
        pltpu.sync_copy(zbuf, cnt_sh.at[pl.ds(base, zbuf.shape[0])])
        pltpu.sync_copy(zbuf, sum_sh.at[pl.ds(base, zbuf.shape[0])])

    plsc.subcore_barrier()

    def _issue_in(w, b):
        base = wid * SH + w * W
        do = pltpu.async_copy(out_hbm.at[pl.ds(base, W)], o_v[b],
                              sem_in.at[b])
        dt = pltpu.async_copy(tgt_hbm.at[pl.ds(base, W)], t_v[b],
                              sem_in.at[b])
        return do, dt

    def _compute(b, gacc):
        def _elem(j, acc):
            o = o_v[b][pl.ds(j * L, L)]
            t = t_v[b][pl.ds(j * L, L)]
            tf = t.astype(jnp.float32)
            e = 1.0 - o * (2.0 * tf - 1.0)
            bits = lax.bitcast_convert_type(e, jnp.int32)
            bin_ = lax.shift_right_logical(bits, SHIFT)
            idx_v[b][pl.ds(j * L, L)] = bin_ + lax.shift_left(t, SECLOG)
            val_v[b][pl.ds(j * L, L)] = jnp.maximum(e, 0.0)
            return acc + tf
        return lax.fori_loop(0, W // L, _elem, gacc, unroll=8)

    def _issue_sc(b):
        dc = pltpu.async_copy(ones_v, cnt_sh.at[idx_v[b]], sem_sc.at[b],
                              add=True)
        ds = pltpu.async_copy(val_v[b], sum_sh.at[idx_v[b]],
                              sem_sc.at[b], add=True)
        return dc, ds

    gacc = zero16
    in_d = [None, None]
    sc_d = [None, None]
    in_d[0] = _issue_in(0, 0)
    for w in range(NWIN):
        b = w & 1
        for d in in_d[b]:
            d.wait()
        if w + 1 < NWIN:
            in_d[1 - b] = _issue_in(w + 1, 1 - b)
        if sc_d[b] is not None:
            for d in sc_d[b]:
                d.wait()
        gacc = _compute(b, gacc)
        sc_d[b] = _issue_sc(b)
    for pair in sc_d:
        if pair is not None:
            for d in pair:
                d.wait()

    gbuf[...] = gacc
    pltpu.sync_copy(gbuf, cnt_sh.at[pl.ds(SEC + GBASE + sid * L, L)])

    plsc.subcore_barrier()

    pltpu.sync_copy(cnt_sh.at[pl.ds(sid * zslice, zslice)],
                    cnt_out.at[cid, pl.ds(sid * zslice, zslice)])
    pltpu.sync_copy(sum_sh.at[pl.ds(sid * zslice, zslice)],
                    sum_out.at[cid, pl.ds(sid * zslice, zslice)])
    pltpu.sync_copy(cnt_sh.at[pl.ds(SEC + sid * zslice, zslice)],
                    cnt_out.at[cid, pl.ds(HSIZE + sid * zslice, zslice)])
    pltpu.sync_copy(sum_sh.at[pl.ds(SEC + sid * zslice, zslice)],
                    sum_out.at[cid, pl.ds(HSIZE + sid * zslice, zslice)])


def _k2_body(cnt_hbm, sum_hbm, loss_out,
             mn_v, pn_v, sn_v, sp_v, buf16, exch_v, exch2_v, out_v, exch_sh):
    sid = lax.axis_index("s")
    iota = lax.iota(jnp.int32, L)
    lane0 = iota == 0
    b0 = HSIZE - (sid + 1) * CHUNK

    def _load_merged(hbm, sec, dst):
        pltpu.sync_copy(hbm.at[0, pl.ds(sec * HSIZE + b0, CHUNK)], dst)

    _load_merged(cnt_hbm, 0, mn_v)
    _load_merged(cnt_hbm, 1, pn_v)
    _load_merged(sum_hbm, 0, sn_v)
    _load_merged(sum_hbm, 1, sp_v)

    pltpu.sync_copy(cnt_hbm.at[1, pl.ds(0 * HSIZE + b0, CHUNK)], exch_v)
    @pl.loop(0, CHUNK // L, unroll=4)
    def _add_mn(j):
        s = pl.ds(j * L, L)
        mn_v[s] += exch_v[s]
    pltpu.sync_copy(cnt_hbm.at[1, pl.ds(1 * HSIZE + b0, CHUNK)], exch_v)
    @pl.loop(0, CHUNK // L, unroll=4)
    def _add_pn(j):
        s = pl.ds(j * L, L)
        pn_v[s] += exch_v[s]
    pltpu.sync_copy(sum_hbm.at[1, pl.ds(0 * HSIZE + b0, CHUNK)], exch_v)
    @pl.loop(0, CHUNK // L, unroll=4)
    def _add_sn(j):
        s = pl.ds(j * L, L)
        sn_v[s] += exch_v[s]
    pltpu.sync_copy(sum_hbm.at[1, pl.ds(1 * HSIZE + b0, CHUNK)], exch_v)
    @pl.loop(0, CHUNK // L, unroll=4)
    def _add_sp(j):
        s = pl.ds(j * L, L)
        sp_v[s] += exch_v[s]

    def _pass_a(j, carry):
        accP, accM, accA = carry
        s = pl.ds(j * L, L)
        binv = b0 + j * L + iota
        real = binv < NBINS
        pv = pn_v[s]
        mv = mn_v[s]
        accP += jnp.where(real, pv, 0.0)
        accA += jnp.where(real, 0.0, pv)
        accM += jnp.where(real, mv, 0.0)
        return accP, accM, accA

    z = jnp.zeros((L,), jnp.float32)
    accP, accM, accA = lax.fori_loop(0, CHUNK // L, _pass_a, (z, z, z))
    sumP = jnp.sum(accP)
    sumM = jnp.sum(accM)
    sumA = jnp.sum(accA)

    row = (jnp.where(lane0, sumP, 0.0)
           + jnp.where(iota == 1, sumM, 0.0)
           + jnp.where(iota == 2, sumA, 0.0))
    buf16[...] = row
    pltpu.sync_copy(buf16, exch_sh.at[pl.ds(sid * L, L)])
    plsc.subcore_barrier()
    pltpu.sync_copy(exch_sh, exch2_v)

    tot = jnp.zeros((L,), jnp.float32)
    pre = jnp.zeros((L,), jnp.float32)
    for j in range(NS):
        rj = exch2_v[pl.ds(j * L, L)]
        tot += rj
        pre += jnp.where(j < sid, rj, 0.0)
    G = jnp.sum(jnp.where(iota == 2, tot, 0.0))
    c0_start = jnp.sum(jnp.where(lane0, pre, 0.0))
    n0_start = jnp.sum(jnp.where(iota == 1, pre, 0.0))

    def _pass_b(j, carry):
        c0r, n0r, acc = carry
        jj = CHUNK // L - 1 - j
        s = pl.ds(jj * L, L)
        binv = b0 + jj * L + iota
        real = binv < NBINS
        p = jnp.where(real, pn_v[s], 0.0)
        m = jnp.where(real, mn_v[s], 0.0)
        Sp = jnp.where(real, sp_v[s], 0.0)
        Sn = jnp.where(real, sn_v[s], 0.0)
        p = lax.rev(p, (0,))
        m = lax.rev(m, (0,))
        Sp = lax.rev(Sp, (0,))
        Sn = lax.rev(Sn, (0,))
        cin_p = plsc.cumsum(p)
        cin_m = plsc.cumsum(m)
        c0 = c0r + cin_p - p
        n0 = n0r + cin_m - m
        d1 = jnp.maximum(G + n0, 1.0)
        d2 = jnp.maximum(G + n0 + m, 1.0)
        acc = acc + Sp / d1 + Sn * (G - c0 - p) / (d1 * d2)
        return (c0r + jnp.sum(p), n0r + jnp.sum(m), acc)

    c0r, n0r, acc = lax.fori_loop(
        0, CHUNK // L, _pass_b,
        (c0_start, n0_start, jnp.zeros((L,), jnp.float32)))
    partial = jnp.sum(acc)

    buf16[...] = jnp.where(lane0, partial, 0.0)
    pltpu.sync_copy(buf16, exch_sh.at[pl.ds(sid * L, L)])
    plsc.subcore_barrier()

    @pl.when(sid == 0)
    def _final():
        pltpu.sync_copy(exch_sh, exch2_v)
        total = jnp.zeros((L,), jnp.float32)
        for j in range(NS):
            total += exch2_v[pl.ds(j * L, L)]
        out_v[...] = total
        pltpu.sync_copy(out_v, loss_out)


@jax.jit
def kernel(outputs, targets):
    targets = targets.astype(jnp.int32)

    mesh1 = plsc.VectorSubcoreMesh(
        core_axis_name="c", subcore_axis_name="s",
        num_cores=NC, num_subcores=NS)
    k1 = pl.kernel(
        _k1_body,
        out_type=(jax.ShapeDtypeStruct((NC, 2 * HSIZE), jnp.float32),
                  jax.ShapeDtypeStruct((NC, 2 * HSIZE), jnp.float32)),
        mesh=mesh1,
        compiler_params=pltpu.CompilerParams(needs_layout_passes=False),
        scratch_types=[
            pltpu.VMEM((W,), jnp.float32),
            pltpu.VMEM((W,), jnp.float32),
            pltpu.VMEM((W,), jnp.int32),
            pltpu.VMEM((W,), jnp.int32),
            pltpu.VMEM((W,), jnp.int32),
            pltpu.VMEM((W,), jnp.int32),
            pltpu.VMEM((W,), jnp.float32),
            pltpu.VMEM((W,), jnp.float32),
            pltpu.VMEM((W,), jnp.float32),
            pltpu.VMEM((2048,), jnp.float32),
            pltpu.VMEM((L,), jnp.float32),
            pltpu.VMEM_SHARED((2 * SEC,), jnp.float32),
            pltpu.VMEM_SHARED((2 * SEC,), jnp.float32),
            pltpu.SemaphoreType.DMA((2,)),
            pltpu.SemaphoreType.DMA((2,)),
        ],
    )
    cnt, sm = k1(outputs, targets)

    mesh2 = plsc.VectorSubcoreMesh(
        core_axis_name="c", subcore_axis_name="s",
        num_cores=1, num_subcores=NS)
    k2 = pl.kernel(
        _k2_body,
        out_type=jax.ShapeDtypeStruct((L,), jnp.float32),
        mesh=mesh2,
        compiler_params=pltpu.CompilerParams(needs_layout_passes=False),
        scratch_types=[
            pltpu.VMEM((CHUNK,), jnp.float32),
            pltpu.VMEM((CHUNK,), jnp.float32),
            pltpu.VMEM((CHUNK,), jnp.float32),
            pltpu.VMEM((CHUNK,), jnp.float32),
            pltpu.VMEM((L,), jnp.float32),
            pltpu.VMEM((CHUNK,), jnp.float32),
            pltpu.VMEM((NS * L,), jnp.float32),
            pltpu.VMEM((L,), jnp.float32),
            pltpu.VMEM_SHARED((NS * L,), jnp.float32),
        ],
    )
    loss16 = k2(cnt, sm)
    return loss16[0]


# speedup vs baseline: 1.6421x; 1.6421x over previous
import jax
import jax.numpy as jnp
from jax import lax
from jax.experimental import pallas as pl
from jax.experimental.pallas import tpu as pltpu
from jax.experimental.pallas import tpu_sc as plsc

P = 4194304
NC = 2
NS = 16
NW = NC * NS
L = 16
M = 5
SHIFT = 23 - M
NBINS = 255 << M
HSIZE = 256 << M
SEC = 2 * HSIZE
SECLOG = SEC.bit_length() - 1
GBASE = NBINS
SH = P // NW
W = 8192
NWIN = SH // W
CHUNK = HSIZE // NS


def _k1_body(out_hbm, tgt_hbm, hist_out,
             o_v0, o_v1, t_v0, t_v1, cnt_t, sum_t, sem_in):
    o_v = (o_v0, o_v1)
    t_v = (t_v0, t_v1)
    cid = lax.axis_index("c")
    sid = lax.axis_index("s")
    wid = cid * NS + sid
    zero16 = jnp.zeros((L,), jnp.float32)
    one16 = jnp.ones((L,), jnp.float32)

    @pl.loop(0, 2 * SEC // L)
    def _zero(j):
        cnt_t[pl.ds(j * L, L)] = zero16
        sum_t[pl.ds(j * L, L)] = zero16

    def _issue_in(w, b):
        base = wid * SH + w * W
        pltpu.async_copy(out_hbm.at[pl.ds(base, W)], o_v[b], sem_in.at[b])
        pltpu.async_copy(tgt_hbm.at[pl.ds(base, W)], t_v[b], sem_in.at[b])

    def _drain_in(b):
        pltpu.make_async_copy(out_hbm.at[pl.ds(0, W)], o_v[b],
                              sem_in.at[b]).wait()
        pltpu.make_async_copy(tgt_hbm.at[pl.ds(0, W)], t_v[b],
                              sem_in.at[b]).wait()

    def _compute(b, gacc):
        def _elem(j, acc):
            o = o_v[b][pl.ds(j * L, L)]
            t = t_v[b][pl.ds(j * L, L)]
            tf = t.astype(jnp.float32)
            e = 1.0 - o * (2.0 * tf - 1.0)
            bits = lax.bitcast_convert_type(e, jnp.int32)
            bin_ = lax.shift_right_logical(bits, SHIFT)
            idx = bin_ + lax.shift_left(t, SECLOG)
            plsc.addupdate_scatter(cnt_t, [idx], one16)
            plsc.addupdate_scatter(sum_t, [idx], jnp.maximum(e, 0.0))
            return acc + tf
        return plsc.parallel_loop(0, W // L, unroll=4, carry=gacc)(_elem)

    _issue_in(0, 0)
    _issue_in(1, 1)

    def _pair(k, gacc):
        _drain_in(0)
        gacc = _compute(0, gacc)

        @pl.when(k < NWIN // 2 - 1)
        def _i0():
            _issue_in(2 * k + 2, 0)

        _drain_in(1)
        gacc = _compute(1, gacc)

        @pl.when(k < NWIN // 2 - 1)
        def _i1():
            _issue_in(2 * k + 3, 1)

        return gacc

    gacc = lax.fori_loop(0, NWIN // 2, _pair, zero16)

    cnt_t[pl.ds(SEC + GBASE, L)] = gacc

    pltpu.sync_copy(cnt_t.at[pl.ds(0, HSIZE)], hist_out.at[wid, 0])
    pltpu.sync_copy(cnt_t.at[pl.ds(SEC, HSIZE)], hist_out.at[wid, 1])
    pltpu.sync_copy(sum_t.at[pl.ds(0, HSIZE)], hist_out.at[wid, 2])
    pltpu.sync_copy(sum_t.at[pl.ds(SEC, HSIZE)], hist_out.at[wid, 3])


def _k2_body(hist_hbm, loss_out,
             mn_v, pn_v, sn_v, sp_v,
             sa0, sa1, sa2, sa3, sb0, sb1, sb2, sb3,
             buf16, exch2_v, out_v, exch_sh, sem_st):
    sid = lax.axis_index("s")
    iota = lax.iota(jnp.int32, L)
    lane0 = iota == 0
    b0 = HSIZE - (sid + 1) * CHUNK
    zero16 = jnp.zeros((L,), jnp.float32)

    dsts = (mn_v, pn_v, sn_v, sp_v)
    stages = ((sa0, sa1, sa2, sa3), (sb0, sb1, sb2, sb3))

    @pl.loop(0, CHUNK // L)
    def _zdst(j):
        s = pl.ds(j * L, L)
        for dst in dsts:
            dst[s] = zero16

    def _issue(r, b):
        for c in range(4):
            pltpu.async_copy(hist_hbm.at[r, c, pl.ds(b0, CHUNK)],
                             stages[b][c], sem_st.at[b])

    def _drain_add(b):
        for c in range(4):
            pltpu.make_async_copy(hist_hbm.at[0, c, pl.ds(b0, CHUNK)],
                                  stages[b][c], sem_st.at[b]).wait()

        @pl.loop(0, CHUNK // L, unroll=2)
        def _acc(j):
            s = pl.ds(j * L, L)
            for c, dst in enumerate(dsts):
                dst[s] += stages[b][c][s]

    _issue(0, 0)
    _issue(1, 1)

    def _mpair(k, carry):
        _drain_add(0)

        @pl.when(k < NW // 2 - 1)
        def _ia():
            _issue(2 * k + 2, 0)

        _drain_add(1)

        @pl.when(k < NW // 2 - 1)
        def _ib():
            _issue(2 * k + 3, 1)

        return carry

    lax.fori_loop(0, NW // 2, _mpair, 0)

    def _pass_a(j, carry):
        accP, accM, accA = carry
        s = pl.ds(j * L, L)
        binv = b0 + j * L + iota
        real = binv < NBINS
        pv = pn_v[s]
        mv = mn_v[s]
        accP += jnp.where(real, pv, 0.0)
        accA += jnp.where(real, 0.0, pv)
        accM += jnp.where(real, mv, 0.0)
        return accP, accM, accA

    accP, accM, accA = lax.fori_loop(0, CHUNK // L, _pass_a,
                                     (zero16, zero16, zero16))
    sumP = jnp.sum(accP)
    sumM = jnp.sum(accM)
    sumA = jnp.sum(accA)

    row = (jnp.where(lane0, sumP, 0.0)
           + jnp.where(iota == 1, sumM, 0.0)
           + jnp.where(iota == 2, sumA, 0.0))
    buf16[...] = row
    pltpu.sync_copy(buf16, exch_sh.at[pl.ds(sid * L, L)])
    plsc.subcore_barrier()
    pltpu.sync_copy(exch_sh, exch2_v)

    tot = zero16
    pre = zero16
    for j in range(NS):
        rj = exch2_v[pl.ds(j * L, L)]
        tot += rj
        pre += jnp.where(j < sid, rj, 0.0)
    G = jnp.sum(jnp.where(iota == 2, tot, 0.0))
    c0_start = jnp.sum(jnp.where(lane0, pre, 0.0))
    n0_start = jnp.sum(jnp.where(iota == 1, pre, 0.0))

    def _pass_b(j, carry):
        c0r, n0r, acc = carry
        jj = CHUNK // L - 1 - j
        s = pl.ds(jj * L, L)
        binv = b0 + jj * L + iota
        real = binv < NBINS
        p = jnp.where(real, pn_v[s], 0.0)
        m = jnp.where(real, mn_v[s], 0.0)
        Sp = jnp.where(real, sp_v[s], 0.0)
        Sn = jnp.where(real, sn_v[s], 0.0)
        p = lax.rev(p, (0,))
        m = lax.rev(m, (0,))
        Sp = lax.rev(Sp, (0,))
        Sn = lax.rev(Sn, (0,))
        cin_p = plsc.cumsum(p)
        cin_m = plsc.cumsum(m)
        c0 = c0r + cin_p - p
        n0 = n0r + cin_m - m
        d1 = jnp.maximum(G + n0, 1.0)
        d2 = jnp.maximum(G + n0 + m, 1.0)
        acc = acc + Sp / d1 + Sn * (G - c0 - p) / (d1 * d2)
        return (c0r + jnp.sum(p), n0r + jnp.sum(m), acc)

    c0r, n0r, acc = lax.fori_loop(0, CHUNK // L, _pass_b,
                                  (c0_start, n0_start, zero16))
    partial = jnp.sum(acc)

    buf16[...] = jnp.where(lane0, partial, 0.0)
    pltpu.sync_copy(buf16, exch_sh.at[pl.ds(sid * L, L)])
    plsc.subcore_barrier()

    @pl.when(sid == 0)
    def _final():
        pltpu.sync_copy(exch_sh, exch2_v)
        total = jnp.zeros((L,), jnp.float32)
        for j in range(NS):
            total += exch2_v[pl.ds(j * L, L)]
        out_v[...] = total
        pltpu.sync_copy(out_v, loss_out)


@jax.jit
def kernel(outputs, targets):
    targets = targets.astype(jnp.int32)

    mesh1 = plsc.VectorSubcoreMesh(
        core_axis_name="c", subcore_axis_name="s",
        num_cores=NC, num_subcores=NS)
    k1 = pl.kernel(
        _k1_body,
        out_type=jax.ShapeDtypeStruct((NW, 4, HSIZE), jnp.float32),
        mesh=mesh1,
        compiler_params=pltpu.CompilerParams(needs_layout_passes=False),
        scratch_types=[
            pltpu.VMEM((W,), jnp.float32),
            pltpu.VMEM((W,), jnp.float32),
            pltpu.VMEM((W,), jnp.int32),
            pltpu.VMEM((W,), jnp.int32),
            pltpu.VMEM((2 * SEC,), jnp.float32),
            pltpu.VMEM((2 * SEC,), jnp.float32),
            pltpu.SemaphoreType.DMA((2,)),
        ],
    )
    hist = k1(outputs, targets)

    mesh2 = plsc.VectorSubcoreMesh(
        core_axis_name="c", subcore_axis_name="s",
        num_cores=1, num_subcores=NS)
    k2 = pl.kernel(
        _k2_body,
        out_type=jax.ShapeDtypeStruct((L,), jnp.float32),
        mesh=mesh2,
        compiler_params=pltpu.CompilerParams(needs_layout_passes=False),
        scratch_types=[
            pltpu.VMEM((CHUNK,), jnp.float32),
            pltpu.VMEM((CHUNK,), jnp.float32),
            pltpu.VMEM((CHUNK,), jnp.float32),
            pltpu.VMEM((CHUNK,), jnp.float32),
            pltpu.VMEM((CHUNK,), jnp.float32),
            pltpu.VMEM((CHUNK,), jnp.float32),
            pltpu.VMEM((CHUNK,), jnp.float32),
            pltpu.VMEM((CHUNK,), jnp.float32),
            pltpu.VMEM((CHUNK,), jnp.float32),
            pltpu.VMEM((CHUNK,), jnp.float32),
            pltpu.VMEM((CHUNK,), jnp.float32),
            pltpu.VMEM((CHUNK,), jnp.float32),
            pltpu.VMEM((L,), jnp.float32),
            pltpu.VMEM((NS * L,), jnp.float32),
            pltpu.VMEM((L,), jnp.float32),
            pltpu.VMEM_SHARED((NS * L,), jnp.float32),
            pltpu.SemaphoreType.DMA((2,)),
        ],
    )
    loss16 = k2(hist)
    return loss16[0]


# speedup vs baseline: 1.7719x; 1.0790x over previous
import jax
import jax.numpy as jnp
from jax import lax
from jax.experimental import pallas as pl
from jax.experimental.pallas import tpu as pltpu
from jax.experimental.pallas import tpu_sc as plsc

P = 4194304
NC = 2
NS = 16
NW = NC * NS
L = 16
M = 5
SHIFT = 23 - M
NBINS = 255 << M
HSIZE = 256 << M
SEC = 2 * HSIZE
SECLOG = SEC.bit_length() - 1
GBASE = NBINS
SH = P // NW
W = 8192
NWIN = SH // W
CHUNK = HSIZE // NS


def _k1_body(out_hbm, tgt_hbm, hist_out,
             o_v0, o_v1, t_v0, t_v1, cnt_t, sum_t, sem_in):
    o_v = (o_v0, o_v1)
    t_v = (t_v0, t_v1)
    cid = lax.axis_index("c")
    sid = lax.axis_index("s")
    wid = cid * NS + sid
    zero16 = jnp.zeros((L,), jnp.float32)
    one16 = jnp.ones((L,), jnp.float32)

    @pl.loop(0, HSIZE // L)
    def _zero(j):
        cnt_t[pl.ds(j * L, L)] = zero16
        cnt_t[pl.ds(SEC + j * L, L)] = zero16
        sum_t[pl.ds(j * L, L)] = zero16
        sum_t[pl.ds(SEC + j * L, L)] = zero16

    def _issue_in(w, b):
        base = wid * SH + w * W
        pltpu.async_copy(out_hbm.at[pl.ds(base, W)], o_v[b], sem_in.at[b])
        pltpu.async_copy(tgt_hbm.at[pl.ds(base, W)], t_v[b], sem_in.at[b])

    def _drain_in(b):
        pltpu.make_async_copy(out_hbm.at[pl.ds(0, W)], o_v[b],
                              sem_in.at[b]).wait()
        pltpu.make_async_copy(tgt_hbm.at[pl.ds(0, W)], t_v[b],
                              sem_in.at[b]).wait()

    def _compute(b, gacc):
        def _elem(j, acc):
            o = o_v[b][pl.ds(j * L, L)]
            t = t_v[b][pl.ds(j * L, L)]
            tf = t.astype(jnp.float32)
            e = 1.0 - o * (2.0 * tf - 1.0)
            bits = lax.bitcast_convert_type(e, jnp.int32)
            bin_ = lax.shift_right_logical(bits, SHIFT)
            idx = bin_ + lax.shift_left(t, SECLOG)
            plsc.addupdate_scatter(cnt_t, [idx], one16)
            plsc.addupdate_scatter(sum_t, [idx], jnp.maximum(e, 0.0))
            return acc + tf
        return plsc.parallel_loop(0, W // L, unroll=4, carry=gacc)(_elem)

    _issue_in(0, 0)
    _issue_in(1, 1)

    def _pair(k, gacc):
        _drain_in(0)
        gacc = _compute(0, gacc)

        @pl.when(k < NWIN // 2 - 1)
        def _i0():
            _issue_in(2 * k + 2, 0)

        _drain_in(1)
        gacc = _compute(1, gacc)

        @pl.when(k < NWIN // 2 - 1)
        def _i1():
            _issue_in(2 * k + 3, 1)

        return gacc

    gacc = lax.fori_loop(0, NWIN // 2, _pair, zero16)

    cnt_t[pl.ds(SEC + GBASE, L)] = gacc

    pltpu.sync_copy(cnt_t.at[pl.ds(0, HSIZE)], hist_out.at[wid, 0])
    pltpu.sync_copy(cnt_t.at[pl.ds(SEC, HSIZE)], hist_out.at[wid, 1])
    pltpu.sync_copy(sum_t.at[pl.ds(0, HSIZE)], hist_out.at[wid, 2])
    pltpu.sync_copy(sum_t.at[pl.ds(SEC, HSIZE)], hist_out.at[wid, 3])


def _k2_body(hist_hbm, loss_out,
             mn_v, pn_v, sn_v, sp_v,
             sa0, sa1, sa2, sa3, sb0, sb1, sb2, sb3,
             sc0, sc1, sc2, sc3, sd0, sd1, sd2, sd3,
             buf16, exch2_v, out_v, exch_sh, sem_st):
    sid = lax.axis_index("s")
    iota = lax.iota(jnp.int32, L)
    lane0 = iota == 0
    b0 = HSIZE - (sid + 1) * CHUNK
    zero16 = jnp.zeros((L,), jnp.float32)

    dsts = (mn_v, pn_v, sn_v, sp_v)
    stages = ((sa0, sa1, sa2, sa3), (sb0, sb1, sb2, sb3),
              (sc0, sc1, sc2, sc3), (sd0, sd1, sd2, sd3))
    NSET = len(stages)

    @pl.loop(0, CHUNK // L)
    def _zdst(j):
        s = pl.ds(j * L, L)
        for dst in dsts:
            dst[s] = zero16

    def _issue(r, b):
        for c in range(4):
            pltpu.async_copy(hist_hbm.at[r, c, pl.ds(b0, CHUNK)],
                             stages[b][c], sem_st.at[b])

    def _drain_add(b):
        for c in range(4):
            pltpu.make_async_copy(hist_hbm.at[0, c, pl.ds(b0, CHUNK)],
                                  stages[b][c], sem_st.at[b]).wait()

        @pl.loop(0, CHUNK // L, unroll=2)
        def _acc(j):
            s = pl.ds(j * L, L)
            for c, dst in enumerate(dsts):
                dst[s] += stages[b][c][s]

    for b in range(NSET):
        _issue(b, b)

    def _mgroup(k, carry):
        for b in range(NSET):
            _drain_add(b)

            @pl.when(k < NW // NSET - 1)
            def _ia():
                _issue(NSET * k + NSET + b, b)

        return carry

    lax.fori_loop(0, NW // NSET, _mgroup, 0)

    def _pass_a(j, carry):
        accP, accM, accA = carry
        s = pl.ds(j * L, L)
        binv = b0 + j * L + iota
        real = binv < NBINS
        pv = pn_v[s]
        mv = mn_v[s]
        accP += jnp.where(real, pv, 0.0)
        accA += jnp.where(real, 0.0, pv)
        accM += jnp.where(real, mv, 0.0)
        return accP, accM, accA

    accP, accM, accA = lax.fori_loop(0, CHUNK // L, _pass_a,
                                     (zero16, zero16, zero16))
    sumP = jnp.sum(accP)
    sumM = jnp.sum(accM)
    sumA = jnp.sum(accA)

    row = (jnp.where(lane0, sumP, 0.0)
           + jnp.where(iota == 1, sumM, 0.0)
           + jnp.where(iota == 2, sumA, 0.0))
    buf16[...] = row
    pltpu.sync_copy(buf16, exch_sh.at[pl.ds(sid * L, L)])
    plsc.subcore_barrier()
    pltpu.sync_copy(exch_sh, exch2_v)

    tot = zero16
    pre = zero16
    for j in range(NS):
        rj = exch2_v[pl.ds(j * L, L)]
        tot += rj
        pre += jnp.where(j < sid, rj, 0.0)
    G = jnp.sum(jnp.where(iota == 2, tot, 0.0))
    c0_start = jnp.sum(jnp.where(lane0, pre, 0.0))
    n0_start = jnp.sum(jnp.where(iota == 1, pre, 0.0))

    def _pass_b(j, carry):
        c0r, n0r, acc = carry
        jj = CHUNK // L - 1 - j
        s = pl.ds(jj * L, L)
        binv = b0 + jj * L + iota
        real = binv < NBINS
        p = jnp.where(real, pn_v[s], 0.0)
        m = jnp.where(real, mn_v[s], 0.0)
        Sp = jnp.where(real, sp_v[s], 0.0)
        Sn = jnp.where(real, sn_v[s], 0.0)
        p = lax.rev(p, (0,))
        m = lax.rev(m, (0,))
        Sp = lax.rev(Sp, (0,))
        Sn = lax.rev(Sn, (0,))
        cin_p = plsc.cumsum(p)
        cin_m = plsc.cumsum(m)
        c0 = c0r + cin_p - p
        n0 = n0r + cin_m - m
        d1 = jnp.maximum(G + n0, 1.0)
        d2 = jnp.maximum(G + n0 + m, 1.0)
        acc = acc + Sp / d1 + Sn * (G - c0 - p) / (d1 * d2)
        return (c0r + jnp.sum(p), n0r + jnp.sum(m), acc)

    c0r, n0r, acc = lax.fori_loop(0, CHUNK // L, _pass_b,
                                  (c0_start, n0_start, zero16))
    partial = jnp.sum(acc)

    buf16[...] = jnp.where(lane0, partial, 0.0)
    pltpu.sync_copy(buf16, exch_sh.at[pl.ds(sid * L, L)])
    plsc.subcore_barrier()

    @pl.when(sid == 0)
    def _final():
        pltpu.sync_copy(exch_sh, exch2_v)
        total = jnp.zeros((L,), jnp.float32)
        for j in range(NS):
            total += exch2_v[pl.ds(j * L, L)]
        out_v[...] = total
        pltpu.sync_copy(out_v, loss_out)


@jax.jit
def kernel(outputs, targets):
    targets = targets.astype(jnp.int32)

    mesh1 = plsc.VectorSubcoreMesh(
        core_axis_name="c", subcore_axis_name="s",
        num_cores=NC, num_subcores=NS)
    k1 = pl.kernel(
        _k1_body,
        out_type=jax.ShapeDtypeStruct((NW, 4, HSIZE), jnp.float32),
        mesh=mesh1,
        compiler_params=pltpu.CompilerParams(needs_layout_passes=False),
        scratch_types=[
            pltpu.VMEM((W,), jnp.float32),
            pltpu.VMEM((W,), jnp.float32),
            pltpu.VMEM((W,), jnp.int32),
            pltpu.VMEM((W,), jnp.int32),
            pltpu.VMEM((2 * SEC,), jnp.float32),
            pltpu.VMEM((2 * SEC,), jnp.float32),
            pltpu.SemaphoreType.DMA((2,)),
        ],
    )
    hist = k1(outputs, targets)

    mesh2 = plsc.VectorSubcoreMesh(
        core_axis_name="c", subcore_axis_name="s",
        num_cores=1, num_subcores=NS)
    k2 = pl.kernel(
        _k2_body,
        out_type=jax.ShapeDtypeStruct((L,), jnp.float32),
        mesh=mesh2,
        compiler_params=pltpu.CompilerParams(needs_layout_passes=False),
        scratch_types=[
            pltpu.VMEM((CHUNK,), jnp.float32),
            pltpu.VMEM((CHUNK,), jnp.float32),
            pltpu.VMEM((CHUNK,), jnp.float32),
            pltpu.VMEM((CHUNK,), jnp.float32),
            pltpu.VMEM((CHUNK,), jnp.float32),
            pltpu.VMEM((CHUNK,), jnp.float32),
            pltpu.VMEM((CHUNK,), jnp.float32),
            pltpu.VMEM((CHUNK,), jnp.float32),
            pltpu.VMEM((CHUNK,), jnp.float32),
            pltpu.VMEM((CHUNK,), jnp.float32),
            pltpu.VMEM((CHUNK,), jnp.float32),
            pltpu.VMEM((CHUNK,), jnp.float32),
            pltpu.VMEM((CHUNK,), jnp.float32),
            pltpu.VMEM((CHUNK,), jnp.float32),
            pltpu.VMEM((CHUNK,), jnp.float32),
            pltpu.VMEM((CHUNK,), jnp.float32),
            pltpu.VMEM((CHUNK,), jnp.float32),
            pltpu.VMEM((CHUNK,), jnp.float32),
            pltpu.VMEM((CHUNK,), jnp.float32),
            pltpu.VMEM((CHUNK,), jnp.float32),
            pltpu.VMEM((L,), jnp.float32),
            pltpu.VMEM((NS * L,), jnp.float32),
            pltpu.VMEM((L,), jnp.float32),
            pltpu.VMEM_SHARED((NS * L,), jnp.float32),
            pltpu.SemaphoreType.DMA((4,)),
        ],
    )
    loss16 = k2(hist)
    return loss16[0]


# speedup vs baseline: 1.7775x; 1.0032x over previous
import jax
import jax.numpy as jnp
from jax import lax
from jax.experimental import pallas as pl
from jax.experimental.pallas import tpu as pltpu
from jax.experimental.pallas import tpu_sc as plsc

P = 4194304
NC = 2
NS = 16
NW = NC * NS
L = 16
M = 5
SHIFT = 23 - M
NBINS = 255 << M
HSIZE = 256 << M
SEC = 2 * HSIZE
SECLOG = SEC.bit_length() - 1
GBASE = NBINS
SH = P // NW
W = 8192
NWIN = SH // W
CHUNK = HSIZE // NS


def _k1_body(out_hbm, tgt_hbm, hist_out,
             o_v0, o_v1, t_v0, t_v1, cnt_t, sum_t, sem_in):
    o_v = (o_v0, o_v1)
    t_v = (t_v0, t_v1)
    cid = lax.axis_index("c")
    sid = lax.axis_index("s")
    wid = cid * NS + sid
    zero16 = jnp.zeros((L,), jnp.float32)
    one16 = jnp.ones((L,), jnp.float32)

    @pl.loop(0, HSIZE // L)
    def _zero(j):
        cnt_t[pl.ds(j * L, L)] = zero16
        cnt_t[pl.ds(SEC + j * L, L)] = zero16
        sum_t[pl.ds(j * L, L)] = zero16
        sum_t[pl.ds(SEC + j * L, L)] = zero16

    def _issue_in(w, b):
        base = wid * SH + w * W
        pltpu.async_copy(out_hbm.at[pl.ds(base, W)], o_v[b], sem_in.at[b])
        pltpu.async_copy(tgt_hbm.at[pl.ds(base, W)], t_v[b], sem_in.at[b])

    def _drain_in(b):
        pltpu.make_async_copy(out_hbm.at[pl.ds(0, W)], o_v[b],
                              sem_in.at[b]).wait()
        pltpu.make_async_copy(tgt_hbm.at[pl.ds(0, W)], t_v[b],
                              sem_in.at[b]).wait()

    def _compute(b, gacc):
        def _elem(j, acc):
            o = o_v[b][pl.ds(j * L, L)]
            t = t_v[b][pl.ds(j * L, L)]
            tf = t.astype(jnp.float32)
            e = 1.0 - o * (2.0 * tf - 1.0)
            bits = lax.bitcast_convert_type(e, jnp.int32)
            bin_ = lax.shift_right_logical(bits, SHIFT)
            idx = bin_ + lax.shift_left(t, SECLOG)
            plsc.addupdate_scatter(cnt_t, [idx], one16)
            plsc.addupdate_scatter(sum_t, [idx], jnp.maximum(e, 0.0))
            return acc + tf
        return plsc.parallel_loop(0, W // L, unroll=8, carry=gacc)(_elem)

    _issue_in(0, 0)
    _issue_in(1, 1)

    def _pair(k, gacc):
        _drain_in(0)
        gacc = _compute(0, gacc)

        @pl.when(k < NWIN // 2 - 1)
        def _i0():
            _issue_in(2 * k + 2, 0)

        _drain_in(1)
        gacc = _compute(1, gacc)

        @pl.when(k < NWIN // 2 - 1)
        def _i1():
            _issue_in(2 * k + 3, 1)

        return gacc

    gacc = lax.fori_loop(0, NWIN // 2, _pair, zero16)

    cnt_t[pl.ds(SEC + GBASE, L)] = gacc

    pltpu.sync_copy(cnt_t.at[pl.ds(0, HSIZE)], hist_out.at[wid, 0])
    pltpu.sync_copy(cnt_t.at[pl.ds(SEC, HSIZE)], hist_out.at[wid, 1])
    pltpu.sync_copy(sum_t.at[pl.ds(0, HSIZE)], hist_out.at[wid, 2])
    pltpu.sync_copy(sum_t.at[pl.ds(SEC, HSIZE)], hist_out.at[wid, 3])


def _k2_body(hist_hbm, loss_out,
             mn_v, pn_v, sn_v, sp_v,
             sa0, sa1, sa2, sa3, sb0, sb1, sb2, sb3,
             sc0, sc1, sc2, sc3, sd0, sd1, sd2, sd3,
             buf16, exch2_v, out_v, exch_sh, sem_st):
    sid = lax.axis_index("s")
    iota = lax.iota(jnp.int32, L)
    lane0 = iota == 0
    b0 = HSIZE - (sid + 1) * CHUNK
    zero16 = jnp.zeros((L,), jnp.float32)

    dsts = (mn_v, pn_v, sn_v, sp_v)
    stages = ((sa0, sa1, sa2, sa3), (sb0, sb1, sb2, sb3),
              (sc0, sc1, sc2, sc3), (sd0, sd1, sd2, sd3))
    NSET = len(stages)

    @pl.loop(0, CHUNK // L)
    def _zdst(j):
        s = pl.ds(j * L, L)
        for dst in dsts:
            dst[s] = zero16

    def _issue(r, b):
        for c in range(4):
            pltpu.async_copy(hist_hbm.at[r, c, pl.ds(b0, CHUNK)],
                             stages[b][c], sem_st.at[b])

    def _drain_add(b):
        for c in range(4):
            pltpu.make_async_copy(hist_hbm.at[0, c, pl.ds(b0, CHUNK)],
                                  stages[b][c], sem_st.at[b]).wait()

        @pl.loop(0, CHUNK // L, unroll=2)
        def _acc(j):
            s = pl.ds(j * L, L)
            for c, dst in enumerate(dsts):
                dst[s] += stages[b][c][s]

    for b in range(NSET):
        _issue(b, b)

    def _mgroup(k, carry):
        for b in range(NSET):
            _drain_add(b)

            @pl.when(k < NW // NSET - 1)
            def _ia():
                _issue(NSET * k + NSET + b, b)

        return carry

    lax.fori_loop(0, NW // NSET, _mgroup, 0)

    def _pass_a(j, carry):
        accP, accM, accA = carry
        s = pl.ds(j * L, L)
        binv = b0 + j * L + iota
        real = binv < NBINS
        pv = pn_v[s]
        mv = mn_v[s]
        accP += jnp.where(real, pv, 0.0)
        accA += jnp.where(real, 0.0, pv)
        accM += jnp.where(real, mv, 0.0)
        return accP, accM, accA

    accP, accM, accA = lax.fori_loop(0, CHUNK // L, _pass_a,
                                     (zero16, zero16, zero16))
    sumP = jnp.sum(accP)
    sumM = jnp.sum(accM)
    sumA = jnp.sum(accA)

    row = (jnp.where(lane0, sumP, 0.0)
           + jnp.where(iota == 1, sumM, 0.0)
           + jnp.where(iota == 2, sumA, 0.0))
    buf16[...] = row
    pltpu.sync_copy(buf16, exch_sh.at[pl.ds(sid * L, L)])
    plsc.subcore_barrier()
    pltpu.sync_copy(exch_sh, exch2_v)

    tot = zero16
    pre = zero16
    for j in range(NS):
        rj = exch2_v[pl.ds(j * L, L)]
        tot += rj
        pre += jnp.where(j < sid, rj, 0.0)
    G = jnp.sum(jnp.where(iota == 2, tot, 0.0))
    c0_start = jnp.sum(jnp.where(lane0, pre, 0.0))
    n0_start = jnp.sum(jnp.where(iota == 1, pre, 0.0))

    def _pass_b(j, carry):
        c0r, n0r, acc = carry
        jj = CHUNK // L - 1 - j
        s = pl.ds(jj * L, L)
        binv = b0 + jj * L + iota
        real = binv < NBINS
        p = jnp.where(real, pn_v[s], 0.0)
        m = jnp.where(real, mn_v[s], 0.0)
        Sp = jnp.where(real, sp_v[s], 0.0)
        Sn = jnp.where(real, sn_v[s], 0.0)
        p = lax.rev(p, (0,))
        m = lax.rev(m, (0,))
        Sp = lax.rev(Sp, (0,))
        Sn = lax.rev(Sn, (0,))
        cin_p = plsc.cumsum(p)
        cin_m = plsc.cumsum(m)
        c0 = c0r + cin_p - p
        n0 = n0r + cin_m - m
        d1 = jnp.maximum(G + n0, 1.0)
        d2 = jnp.maximum(G + n0 + m, 1.0)
        acc = acc + Sp / d1 + Sn * (G - c0 - p) / (d1 * d2)
        return (c0r + jnp.sum(p), n0r + jnp.sum(m), acc)

    c0r, n0r, acc = lax.fori_loop(0, CHUNK // L, _pass_b,
                                  (c0_start, n0_start, zero16))
    partial = jnp.sum(acc)

    buf16[...] = jnp.where(lane0, partial, 0.0)
    pltpu.sync_copy(buf16, exch_sh.at[pl.ds(sid * L, L)])
    plsc.subcore_barrier()

    @pl.when(sid == 0)
    def _final():
        pltpu.sync_copy(exch_sh, exch2_v)
        total = jnp.zeros((L,), jnp.float32)
        for j in range(NS):
            total += exch2_v[pl.ds(j * L, L)]
        out_v[...] = total
        pltpu.sync_copy(out_v, loss_out)


@jax.jit
def kernel(outputs, targets):
    targets = targets.astype(jnp.int32)

    mesh1 = plsc.VectorSubcoreMesh(
        core_axis_name="c", subcore_axis_name="s",
        num_cores=NC, num_subcores=NS)
    k1 = pl.kernel(
        _k1_body,
        out_type=jax.ShapeDtypeStruct((NW, 4, HSIZE), jnp.float32),
        mesh=mesh1,
        compiler_params=pltpu.CompilerParams(needs_layout_passes=False),
        scratch_types=[
            pltpu.VMEM((W,), jnp.float32),
            pltpu.VMEM((W,), jnp.float32),
            pltpu.VMEM((W,), jnp.int32),
            pltpu.VMEM((W,), jnp.int32),
            pltpu.VMEM((2 * SEC,), jnp.float32),
            pltpu.VMEM((2 * SEC,), jnp.float32),
            pltpu.SemaphoreType.DMA((2,)),
        ],
    )
    hist = k1(outputs, targets)

    mesh2 = plsc.VectorSubcoreMesh(
        core_axis_name="c", subcore_axis_name="s",
        num_cores=1, num_subcores=NS)
    k2 = pl.kernel(
        _k2_body,
        out_type=jax.ShapeDtypeStruct((L,), jnp.float32),
        mesh=mesh2,
        compiler_params=pltpu.CompilerParams(needs_layout_passes=False),
        scratch_types=[
            pltpu.VMEM((CHUNK,), jnp.float32),
            pltpu.VMEM((CHUNK,), jnp.float32),
            pltpu.VMEM((CHUNK,), jnp.float32),
            pltpu.VMEM((CHUNK,), jnp.float32),
            pltpu.VMEM((CHUNK,), jnp.float32),
            pltpu.VMEM((CHUNK,), jnp.float32),
            pltpu.VMEM((CHUNK,), jnp.float32),
            pltpu.VMEM((CHUNK,), jnp.float32),
            pltpu.VMEM((CHUNK,), jnp.float32),
            pltpu.VMEM((CHUNK,), jnp.float32),
            pltpu.VMEM((CHUNK,), jnp.float32),
            pltpu.VMEM((CHUNK,), jnp.float32),
            pltpu.VMEM((CHUNK,), jnp.float32),
            pltpu.VMEM((CHUNK,), jnp.float32),
            pltpu.VMEM((CHUNK,), jnp.float32),
            pltpu.VMEM((CHUNK,), jnp.float32),
            pltpu.VMEM((CHUNK,), jnp.float32),
            pltpu.VMEM((CHUNK,), jnp.float32),
            pltpu.VMEM((CHUNK,), jnp.float32),
            pltpu.VMEM((CHUNK,), jnp.float32),
            pltpu.VMEM((L,), jnp.float32),
            pltpu.VMEM((NS * L,), jnp.float32),
            pltpu.VMEM((L,), jnp.float32),
            pltpu.VMEM_SHARED((NS * L,), jnp.float32),
            pltpu.SemaphoreType.DMA((4,)),
        ],
    )
    loss16 = k2(hist)
    return loss16[0]
